# baseline (speedup 1.0000x reference)
import functools
import math

import jax
import jax.numpy as jnp
from jax import lax
from jax.experimental import pallas as pl
from jax.experimental.pallas import tpu as pltpu
from jax.experimental.pallas import tpu_sc as plsc

N_EMBD = 512
NUM_EXPERTS = 4
D_FF = 4 * N_EMBD
_GELU_C = math.sqrt(2.0 / math.pi)

BT = 1024
BLK = 512
NCOMBO = 6
_COMBO_A = (0, 0, 0, 1, 1, 2)
_COMBO_B = (1, 2, 3, 2, 3, 3)

_NC, _NS, _L = 2, 16, 16
_NW = _NC * _NS
_CH = 128


def _gelu(h):
    return 0.5 * h * (1.0 + jnp.tanh(_GELU_C * (h + 0.044715 * h ** 3)))


def _gate_body(x_ref, Wg_ref, c_ref, r_ref, wa_ref, wb_ref, tot_ref,
               loss_ref, gsum_ref):
    i = pl.program_id(0)
    nb = pl.num_programs(0)
    xb = x_ref[...]
    g = jnp.dot(xb, Wg_ref[...])

    @pl.when(i == 0)
    def _():
        gsum_ref[0, 0] = 0.0
        tot_ref[...] = jnp.zeros_like(tot_ref)

    gsum_ref[0, 0] += jnp.sum(g)

    col = lax.broadcasted_iota(jnp.int32, g.shape, 1)
    m1 = jnp.max(g, axis=1, keepdims=True)
    e1 = jnp.argmax(g, axis=1).astype(jnp.int32)
    gm = jnp.where(col == e1[:, None], -jnp.inf, g)
    m2 = jnp.max(gm, axis=1, keepdims=True)
    e2 = jnp.argmax(gm, axis=1).astype(jnp.int32)
    t = jnp.exp(m2 - m1)[:, 0]
    w1 = 1.0 / (1.0 + t)
    w2 = 1.0 - w1

    a = jnp.minimum(e1, e2)
    b = jnp.maximum(e1, e2)
    wa = jnp.where(e1 < e2, w1, w2)
    wb = jnp.where(e1 < e2, w2, w1)
    c = a * 3 - (a * (a - 1)) // 2 + (b - a - 1)

    ccol = lax.broadcasted_iota(jnp.int32, (BT, NCOMBO), 1)
    onehot = (c[:, None] == ccol).astype(jnp.float32)
    row_i = lax.broadcasted_iota(jnp.int32, (BT, BT), 0)
    col_i = lax.broadcasted_iota(jnp.int32, (BT, BT), 1)
    lt = (col_i < row_i).astype(jnp.float32)
    prev = jnp.dot(lt, onehot)

    tot = tot_ref[...]
    r = jnp.sum((prev + tot) * onehot, axis=1)
    tot_ref[...] = tot + jnp.sum(onehot, axis=0, keepdims=True)

    c_ref[...] = c[None, None, :]
    r_ref[...] = r.astype(jnp.int32)[None, None, :]
    wa_ref[...] = wa[None, None, :]
    wb_ref[...] = wb[None, None, :]

    @pl.when(i == nb - 1)
    def _():
        s = gsum_ref[0, 0] / jnp.float32(nb * BT * NUM_EXPERTS)
        loss_ref[...] = jnp.broadcast_to(s * jnp.log(s + 0.1), (1, 1))


def _sc_mesh():
    return plsc.VectorSubcoreMesh(core_axis_name="c", subcore_axis_name="s",
                                  num_cores=_NC, num_subcores=_NS)


def _run_dispatch(flat, c, r, wa, wb, offp8, T, P):
    per = T // _NW

    @functools.partial(
        pl.kernel, mesh=_sc_mesh(),
        out_type=[
            jax.ShapeDtypeStruct((P, N_EMBD), jnp.float32),
            jax.ShapeDtypeStruct((P,), jnp.float32),
            jax.ShapeDtypeStruct((P,), jnp.float32),
            jax.ShapeDtypeStruct((T,), jnp.int32),
        ],
        scratch_types=[
            pltpu.VMEM((_CH,), jnp.int32),
            pltpu.VMEM((_CH,), jnp.int32),
            pltpu.VMEM((_CH,), jnp.int32),
            pltpu.VMEM((_CH,), jnp.float32),
            pltpu.VMEM((_CH,), jnp.float32),
            pltpu.VMEM((_CH, N_EMBD), jnp.float32),
            pltpu.VMEM((16,), jnp.int32),
            pltpu.SemaphoreType.DMA,
        ],
    )
    def _dispatch(x_hbm, c_hbm, r_hbm, wa_hbm, wb_hbm, offp_hbm,
                  xs_hbm, ws1_hbm, ws2_hbm, pos_hbm,
                  c_v, r_v, p_v, wa_v, wb_v, rows_v, offp_v, sem):
        wid = lax.axis_index("s") * _NC + lax.axis_index("c")
        base = wid * per
        pltpu.sync_copy(offp_hbm, offp_v)
        for jc in range(per // _CH):
            o = base + jc * _CH
            pltpu.sync_copy(c_hbm.at[pl.ds(o, _CH)], c_v)
            pltpu.sync_copy(r_hbm.at[pl.ds(o, _CH)], r_v)
            pltpu.sync_copy(wa_hbm.at[pl.ds(o, _CH)], wa_v)
            pltpu.sync_copy(wb_hbm.at[pl.ds(o, _CH)], wb_v)
            pltpu.sync_copy(x_hbm.at[pl.ds(o, _CH)], rows_v)
            ov = offp_v[...]
            o0 = ov[0]
            o1 = ov[1]
            o2 = ov[2]
            o3 = ov[3]
            o4 = ov[4]
            o5 = ov[5]
            for j in range(_CH // _L):
                sl = pl.ds(j * _L, _L)
                cc = c_v[sl]
                off = jnp.where(
                    cc < 3,
                    jnp.where(cc == 0, o0, jnp.where(cc == 1, o1, o2)),
                    jnp.where(cc == 3, o3, jnp.where(cc == 4, o4, o5)))
                p_v[sl] = off + r_v[sl]
            pltpu.sync_copy(p_v, pos_hbm.at[pl.ds(o, _CH)])
            pltpu.async_copy(rows_v, xs_hbm.at[p_v], sem).wait()
            pltpu.async_copy(wa_v, ws1_hbm.at[p_v], sem).wait()
            pltpu.async_copy(wb_v, ws2_hbm.at[p_v], sem).wait()

    return _dispatch(flat, c, r, wa, wb, offp8)


def _run_combine(y, pos, T):
    per = T // _NW

    @functools.partial(
        pl.kernel, mesh=_sc_mesh(),
        out_type=jax.ShapeDtypeStruct((T, N_EMBD), jnp.float32),
        scratch_types=[
            pltpu.VMEM((_CH,), jnp.int32),
            pltpu.VMEM((_CH, N_EMBD), jnp.float32),
            pltpu.SemaphoreType.DMA,
        ],
    )
    def _combine(ys_hbm, pos_hbm, out_hbm, p_v, rows_v, sem):
        wid = lax.axis_index("s") * _NC + lax.axis_index("c")
        base = wid * per
        for jc in range(per // _CH):
            o = base + jc * _CH
            pltpu.sync_copy(pos_hbm.at[pl.ds(o, _CH)], p_v)
            pltpu.async_copy(ys_hbm.at[p_v], rows_v, sem).wait()
            pltpu.sync_copy(rows_v, out_hbm.at[pl.ds(o, _CH)])

    return _combine(y, pos)


def _ffn_body(bmapA, bmapB, xs_ref, ws1_ref, ws2_ref, W1a_ref, W1b_ref,
              b1a_ref, b1b_ref, W2a_ref, W2b_ref, b2a_ref, b2b_ref, y_ref):
    del bmapA, bmapB
    xb = xs_ref[...]
    ha = _gelu(jnp.dot(xb, W1a_ref[0]) + b1a_ref[0, 0][None, :])
    ya = jnp.dot(ha, W2a_ref[0]) + b2a_ref[0, 0][None, :]
    hb = _gelu(jnp.dot(xb, W1b_ref[0]) + b1b_ref[0, 0][None, :])
    yb = jnp.dot(hb, W2b_ref[0]) + b2b_ref[0, 0][None, :]
    y_ref[...] = ws1_ref[0, 0][:, None] * ya + ws2_ref[0, 0][:, None] * yb


def kernel(x, Wg, W1, b1, W2, b2):
    T = x.shape[0] * x.shape[1]
    flat = x.reshape(T, N_EMBD)
    nb = T // BT
    NB = T // BLK + NCOMBO
    P = NB * BLK

    c3, r3, wa3, wb3, tot, loss = pl.pallas_call(
        _gate_body,
        grid=(nb,),
        in_specs=[
            pl.BlockSpec((BT, N_EMBD), lambda i: (i, 0)),
            pl.BlockSpec((N_EMBD, NUM_EXPERTS), lambda i: (0, 0)),
        ],
        out_specs=[
            pl.BlockSpec((1, 1, BT), lambda i: (i, 0, 0)),
            pl.BlockSpec((1, 1, BT), lambda i: (i, 0, 0)),
            pl.BlockSpec((1, 1, BT), lambda i: (i, 0, 0)),
            pl.BlockSpec((1, 1, BT), lambda i: (i, 0, 0)),
            pl.BlockSpec((1, NCOMBO), lambda i: (0, 0)),
            pl.BlockSpec((1, 1), lambda i: (0, 0)),
        ],
        out_shape=[
            jax.ShapeDtypeStruct((nb, 1, BT), jnp.int32),
            jax.ShapeDtypeStruct((nb, 1, BT), jnp.int32),
            jax.ShapeDtypeStruct((nb, 1, BT), jnp.float32),
            jax.ShapeDtypeStruct((nb, 1, BT), jnp.float32),
            jax.ShapeDtypeStruct((1, NCOMBO), jnp.float32),
            jax.ShapeDtypeStruct((1, 1), jnp.float32),
        ],
        scratch_shapes=[pltpu.SMEM((1, 1), jnp.float32)],
    )(flat, Wg)

    c = c3.reshape(T)
    r = r3.reshape(T)
    wa = wa3.reshape(T)
    wb = wb3.reshape(T)

    n = tot.reshape(NCOMBO).astype(jnp.int32)
    nblk = (n + BLK - 1) // BLK
    cumblk = jnp.cumsum(nblk)
    offp = (jnp.concatenate([jnp.zeros((1,), jnp.int32), cumblk[:-1]])
            * BLK).astype(jnp.int32)
    offp8 = jnp.pad(offp, (0, 16 - NCOMBO))
    bidx = jnp.arange(NB, dtype=jnp.int32)
    cb = jnp.minimum(
        jnp.sum((bidx[:, None] >= cumblk[None, :]).astype(jnp.int32), axis=1),
        NCOMBO - 1)
    bmapA = jnp.asarray(_COMBO_A, jnp.int32)[cb]
    bmapB = jnp.asarray(_COMBO_B, jnp.int32)[cb]

    xs, ws1, ws2, pos = _run_dispatch(flat, c, r, wa, wb, offp8, T, P)
    y = pl.pallas_call(
        _ffn_body,
        grid_spec=pltpu.PrefetchScalarGridSpec(
            num_scalar_prefetch=2,
            grid=(NB,),
            in_specs=[
                pl.BlockSpec((BLK, N_EMBD), lambda i, A, B: (i, 0)),
                pl.BlockSpec((1, 1, BLK), lambda i, A, B: (i, 0, 0)),
                pl.BlockSpec((1, 1, BLK), lambda i, A, B: (i, 0, 0)),
                pl.BlockSpec((1, N_EMBD, D_FF), lambda i, A, B: (A[i], 0, 0)),
                pl.BlockSpec((1, N_EMBD, D_FF), lambda i, A, B: (B[i], 0, 0)),
                pl.BlockSpec((1, 1, D_FF), lambda i, A, B: (A[i], 0, 0)),
                pl.BlockSpec((1, 1, D_FF), lambda i, A, B: (B[i], 0, 0)),
                pl.BlockSpec((1, D_FF, N_EMBD), lambda i, A, B: (A[i], 0, 0)),
                pl.BlockSpec((1, D_FF, N_EMBD), lambda i, A, B: (B[i], 0, 0)),
                pl.BlockSpec((1, 1, N_EMBD), lambda i, A, B: (A[i], 0, 0)),
                pl.BlockSpec((1, 1, N_EMBD), lambda i, A, B: (B[i], 0, 0)),
            ],
            out_specs=pl.BlockSpec((BLK, N_EMBD), lambda i, A, B: (i, 0)),
        ),
        out_shape=jax.ShapeDtypeStruct((P, N_EMBD), jnp.float32),
    )(bmapA, bmapB, xs, ws1.reshape(NB, 1, BLK), ws2.reshape(NB, 1, BLK),
      W1, W1, b1.reshape(NUM_EXPERTS, 1, D_FF), b1.reshape(NUM_EXPERTS, 1, D_FF),
      W2, W2, b2.reshape(NUM_EXPERTS, 1, N_EMBD),
      b2.reshape(NUM_EXPERTS, 1, N_EMBD))

    out = _run_combine(y, pos, T)
    return out.reshape(x.shape), loss[0, 0]


# speedup vs baseline: 1.0255x; 1.0255x over previous
import functools
import math

import jax
import jax.numpy as jnp
from jax import lax
from jax.experimental import pallas as pl
from jax.experimental.pallas import tpu as pltpu
from jax.experimental.pallas import tpu_sc as plsc

N_EMBD = 512
NUM_EXPERTS = 4
D_FF = 4 * N_EMBD
_GELU_C = math.sqrt(2.0 / math.pi)

BT = 1024
BLK = 512
NCOMBO = 6
_COMBO_A = (0, 0, 0, 1, 1, 2)
_COMBO_B = (1, 2, 3, 2, 3, 3)

_NC, _NS, _L = 2, 16, 16
_NW = _NC * _NS
_CH = 64


def _gelu(h):
    return 0.5 * h * (1.0 + jnp.tanh(_GELU_C * (h + 0.044715 * h ** 3)))


def _gate_body(x_ref, Wg_ref, c_ref, r_ref, wa_ref, wb_ref, tot_ref,
               loss_ref, gsum_ref, lt_ref):
    i = pl.program_id(0)
    nb = pl.num_programs(0)
    xb = x_ref[...]
    g = jnp.dot(xb, Wg_ref[...])

    @pl.when(i == 0)
    def _():
        gsum_ref[0, 0] = 0.0
        tot_ref[...] = jnp.zeros_like(tot_ref)

    gsum_ref[0, 0] += jnp.sum(g)

    col = lax.broadcasted_iota(jnp.int32, g.shape, 1)
    m1 = jnp.max(g, axis=1, keepdims=True)
    e1 = jnp.argmax(g, axis=1).astype(jnp.int32)
    gm = jnp.where(col == e1[:, None], -jnp.inf, g)
    m2 = jnp.max(gm, axis=1, keepdims=True)
    e2 = jnp.argmax(gm, axis=1).astype(jnp.int32)
    t = jnp.exp(m2 - m1)[:, 0]
    w1 = 1.0 / (1.0 + t)
    w2 = 1.0 - w1

    a = jnp.minimum(e1, e2)
    b = jnp.maximum(e1, e2)
    wa = jnp.where(e1 < e2, w1, w2)
    wb = jnp.where(e1 < e2, w2, w1)
    c = a * 3 - (a * (a - 1)) // 2 + (b - a - 1)

    ccol = lax.broadcasted_iota(jnp.int32, (BT, NCOMBO), 1)
    onehot = (c[:, None] == ccol).astype(jnp.float32)

    @pl.when(i == 0)
    def _():
        row_i = lax.broadcasted_iota(jnp.int32, (BT, BT), 0)
        col_i = lax.broadcasted_iota(jnp.int32, (BT, BT), 1)
        lt_ref[...] = (col_i < row_i).astype(jnp.float32)

    prev = jnp.dot(lt_ref[...], onehot)

    tot = tot_ref[...]
    r = jnp.sum((prev + tot) * onehot, axis=1)
    tot_ref[...] = tot + jnp.sum(onehot, axis=0, keepdims=True)

    c_ref[...] = c[None, None, :]
    r_ref[...] = r.astype(jnp.int32)[None, None, :]
    wa_ref[...] = wa[None, None, :]
    wb_ref[...] = wb[None, None, :]

    @pl.when(i == nb - 1)
    def _():
        s = gsum_ref[0, 0] / jnp.float32(nb * BT * NUM_EXPERTS)
        loss_ref[...] = jnp.broadcast_to(s * jnp.log(s + 0.1), (1, 1))


def _sc_mesh():
    return plsc.VectorSubcoreMesh(core_axis_name="c", subcore_axis_name="s",
                                  num_cores=_NC, num_subcores=_NS)


def _run_dispatch(flat, c, r, wa, wb, offp8, T, P):
    per = T // _NW

    @functools.partial(
        pl.kernel, mesh=_sc_mesh(),
        out_type=[
            jax.ShapeDtypeStruct((P, N_EMBD), jnp.float32),
            jax.ShapeDtypeStruct((P,), jnp.float32),
            jax.ShapeDtypeStruct((P,), jnp.float32),
            jax.ShapeDtypeStruct((T,), jnp.int32),
        ],
        scratch_types=[
            pltpu.VMEM((_CH,), jnp.int32),
            pltpu.VMEM((_CH,), jnp.int32),
            pltpu.VMEM((_CH,), jnp.int32),
            pltpu.VMEM((_CH,), jnp.int32),
            pltpu.VMEM((_CH,), jnp.float32),
            pltpu.VMEM((_CH,), jnp.float32),
            pltpu.VMEM((_CH,), jnp.float32),
            pltpu.VMEM((_CH,), jnp.float32),
            pltpu.VMEM((_CH, N_EMBD), jnp.float32),
            pltpu.VMEM((_CH, N_EMBD), jnp.float32),
            pltpu.VMEM((16,), jnp.int32),
            pltpu.SemaphoreType.DMA,
            pltpu.SemaphoreType.DMA,
            pltpu.SemaphoreType.DMA,
            pltpu.SemaphoreType.DMA,
        ],
    )
    def _dispatch(x_hbm, c_hbm, r_hbm, wa_hbm, wb_hbm, offp_hbm,
                  xs_hbm, ws1_hbm, ws2_hbm, pos_hbm,
                  c_v, r_v, p_v0, p_v1, wa_v0, wa_v1, wb_v0, wb_v1,
                  rows_v0, rows_v1, offp_v, semi0, semi1, semo0, semo1):
        wid = lax.axis_index("s") * _NC + lax.axis_index("c")
        base = wid * per
        pltpu.sync_copy(offp_hbm, offp_v)
        ov = offp_v[...]
        o0 = ov[0]
        o1 = ov[1]
        o2 = ov[2]
        o3 = ov[3]
        o4 = ov[4]
        o5 = ov[5]
        p_b = (p_v0, p_v1)
        wa_b = (wa_v0, wa_v1)
        wb_b = (wb_v0, wb_v1)
        rows_b = (rows_v0, rows_v1)
        semi_b = (semi0, semi1)
        semo_b = (semo0, semo1)
        nch = per // _CH
        pend_in = [None, None]
        pend_out = [None, None]

        def start_in(jc, bsel):
            o = base + jc * _CH
            pend_in[bsel] = [
                pltpu.async_copy(x_hbm.at[pl.ds(o, _CH)], rows_b[bsel],
                                 semi_b[bsel]),
                pltpu.async_copy(wa_hbm.at[pl.ds(o, _CH)], wa_b[bsel],
                                 semi_b[bsel]),
                pltpu.async_copy(wb_hbm.at[pl.ds(o, _CH)], wb_b[bsel],
                                 semi_b[bsel]),
            ]

        start_in(0, 0)
        for jc in range(nch):
            bsel = jc % 2
            if jc + 1 < nch:
                if pend_out[1 - bsel] is not None:
                    for cp in pend_out[1 - bsel]:
                        cp.wait()
                    pend_out[1 - bsel] = None
                start_in(jc + 1, 1 - bsel)
            o = base + jc * _CH
            p_v = p_b[bsel]
            pltpu.sync_copy(c_hbm.at[pl.ds(o, _CH)], c_v)
            pltpu.sync_copy(r_hbm.at[pl.ds(o, _CH)], r_v)
            for j in range(_CH // _L):
                sl = pl.ds(j * _L, _L)
                cc = c_v[sl]
                off = jnp.where(
                    cc < 3,
                    jnp.where(cc == 0, o0, jnp.where(cc == 1, o1, o2)),
                    jnp.where(cc == 3, o3, jnp.where(cc == 4, o4, o5)))
                p_v[sl] = off + r_v[sl]
            pltpu.sync_copy(p_v, pos_hbm.at[pl.ds(o, _CH)])
            for cp in pend_in[bsel]:
                cp.wait()
            pend_in[bsel] = None
            pend_out[bsel] = [
                pltpu.async_copy(rows_b[bsel], xs_hbm.at[p_v], semo_b[bsel]),
                pltpu.async_copy(wa_b[bsel], ws1_hbm.at[p_v], semo_b[bsel]),
                pltpu.async_copy(wb_b[bsel], ws2_hbm.at[p_v], semo_b[bsel]),
            ]
        for bsel in (0, 1):
            if pend_out[bsel] is not None:
                for cp in pend_out[bsel]:
                    cp.wait()

    return _dispatch(flat, c, r, wa, wb, offp8)


def _run_combine(y, pos, T):
    per = T // _NW

    @functools.partial(
        pl.kernel, mesh=_sc_mesh(),
        out_type=jax.ShapeDtypeStruct((T, N_EMBD), jnp.float32),
        scratch_types=[
            pltpu.VMEM((_CH,), jnp.int32),
            pltpu.VMEM((_CH,), jnp.int32),
            pltpu.VMEM((_CH, N_EMBD), jnp.float32),
            pltpu.VMEM((_CH, N_EMBD), jnp.float32),
            pltpu.SemaphoreType.DMA,
            pltpu.SemaphoreType.DMA,
            pltpu.SemaphoreType.DMA,
            pltpu.SemaphoreType.DMA,
        ],
    )
    def _combine(ys_hbm, pos_hbm, out_hbm, p_v0, p_v1, rows_v0, rows_v1,
                 semg0, semg1, semo0, semo1):
        wid = lax.axis_index("s") * _NC + lax.axis_index("c")
        base = wid * per
        p_b = (p_v0, p_v1)
        rows_b = (rows_v0, rows_v1)
        semg_b = (semg0, semg1)
        semo_b = (semo0, semo1)
        nch = per // _CH
        pend_g = [None, None]
        pend_o = [None, None]

        def start_gather(jc, bsel):
            o = base + jc * _CH
            pltpu.sync_copy(pos_hbm.at[pl.ds(o, _CH)], p_b[bsel])
            pend_g[bsel] = pltpu.async_copy(ys_hbm.at[p_b[bsel]],
                                            rows_b[bsel], semg_b[bsel])

        start_gather(0, 0)
        for jc in range(nch):
            bsel = jc % 2
            if jc + 1 < nch:
                if pend_o[1 - bsel] is not None:
                    pend_o[1 - bsel].wait()
                    pend_o[1 - bsel] = None
                start_gather(jc + 1, 1 - bsel)
            o = base + jc * _CH
            pend_g[bsel].wait()
            pend_g[bsel] = None
            pend_o[bsel] = pltpu.async_copy(
                rows_b[bsel], out_hbm.at[pl.ds(o, _CH)], semo_b[bsel])
        for bsel in (0, 1):
            if pend_o[bsel] is not None:
                pend_o[bsel].wait()

    return _combine(y, pos)


def _ffn_body(bmapA, bmapB, xs_ref, ws1_ref, ws2_ref, W1a_ref, W1b_ref,
              b1a_ref, b1b_ref, W2a_ref, W2b_ref, b2a_ref, b2b_ref, y_ref):
    del bmapA, bmapB
    xb = xs_ref[...]
    ha = _gelu(jnp.dot(xb, W1a_ref[0]) + b1a_ref[0, 0][None, :])
    ya = jnp.dot(ha, W2a_ref[0]) + b2a_ref[0, 0][None, :]
    hb = _gelu(jnp.dot(xb, W1b_ref[0]) + b1b_ref[0, 0][None, :])
    yb = jnp.dot(hb, W2b_ref[0]) + b2b_ref[0, 0][None, :]
    y_ref[...] = ws1_ref[0, 0][:, None] * ya + ws2_ref[0, 0][:, None] * yb


def kernel(x, Wg, W1, b1, W2, b2):
    T = x.shape[0] * x.shape[1]
    flat = x.reshape(T, N_EMBD)
    nb = T // BT
    NB = T // BLK + NCOMBO
    P = NB * BLK

    c3, r3, wa3, wb3, tot, loss = pl.pallas_call(
        _gate_body,
        grid=(nb,),
        in_specs=[
            pl.BlockSpec((BT, N_EMBD), lambda i: (i, 0)),
            pl.BlockSpec((N_EMBD, NUM_EXPERTS), lambda i: (0, 0)),
        ],
        out_specs=[
            pl.BlockSpec((1, 1, BT), lambda i: (i, 0, 0)),
            pl.BlockSpec((1, 1, BT), lambda i: (i, 0, 0)),
            pl.BlockSpec((1, 1, BT), lambda i: (i, 0, 0)),
            pl.BlockSpec((1, 1, BT), lambda i: (i, 0, 0)),
            pl.BlockSpec((1, NCOMBO), lambda i: (0, 0)),
            pl.BlockSpec((1, 1), lambda i: (0, 0)),
        ],
        out_shape=[
            jax.ShapeDtypeStruct((nb, 1, BT), jnp.int32),
            jax.ShapeDtypeStruct((nb, 1, BT), jnp.int32),
            jax.ShapeDtypeStruct((nb, 1, BT), jnp.float32),
            jax.ShapeDtypeStruct((nb, 1, BT), jnp.float32),
            jax.ShapeDtypeStruct((1, NCOMBO), jnp.float32),
            jax.ShapeDtypeStruct((1, 1), jnp.float32),
        ],
        scratch_shapes=[pltpu.SMEM((1, 1), jnp.float32),
                        pltpu.VMEM((BT, BT), jnp.float32)],
    )(flat, Wg)

    c = c3.reshape(T)
    r = r3.reshape(T)
    wa = wa3.reshape(T)
    wb = wb3.reshape(T)

    n = tot.reshape(NCOMBO).astype(jnp.int32)
    nblk = (n + BLK - 1) // BLK
    cumblk = jnp.cumsum(nblk)
    offp = (jnp.concatenate([jnp.zeros((1,), jnp.int32), cumblk[:-1]])
            * BLK).astype(jnp.int32)
    offp8 = jnp.pad(offp, (0, 16 - NCOMBO))
    bidx = jnp.arange(NB, dtype=jnp.int32)
    cb = jnp.minimum(
        jnp.sum((bidx[:, None] >= cumblk[None, :]).astype(jnp.int32), axis=1),
        NCOMBO - 1)
    bmapA = jnp.asarray(_COMBO_A, jnp.int32)[cb]
    bmapB = jnp.asarray(_COMBO_B, jnp.int32)[cb]

    xs, ws1, ws2, pos = _run_dispatch(flat, c, r, wa, wb, offp8, T, P)
    y = pl.pallas_call(
        _ffn_body,
        grid_spec=pltpu.PrefetchScalarGridSpec(
            num_scalar_prefetch=2,
            grid=(NB,),
            in_specs=[
                pl.BlockSpec((BLK, N_EMBD), lambda i, A, B: (i, 0)),
                pl.BlockSpec((1, 1, BLK), lambda i, A, B: (i, 0, 0)),
                pl.BlockSpec((1, 1, BLK), lambda i, A, B: (i, 0, 0)),
                pl.BlockSpec((1, N_EMBD, D_FF), lambda i, A, B: (A[i], 0, 0)),
                pl.BlockSpec((1, N_EMBD, D_FF), lambda i, A, B: (B[i], 0, 0)),
                pl.BlockSpec((1, 1, D_FF), lambda i, A, B: (A[i], 0, 0)),
                pl.BlockSpec((1, 1, D_FF), lambda i, A, B: (B[i], 0, 0)),
                pl.BlockSpec((1, D_FF, N_EMBD), lambda i, A, B: (A[i], 0, 0)),
                pl.BlockSpec((1, D_FF, N_EMBD), lambda i, A, B: (B[i], 0, 0)),
                pl.BlockSpec((1, 1, N_EMBD), lambda i, A, B: (A[i], 0, 0)),
                pl.BlockSpec((1, 1, N_EMBD), lambda i, A, B: (B[i], 0, 0)),
            ],
            out_specs=pl.BlockSpec((BLK, N_EMBD), lambda i, A, B: (i, 0)),
        ),
        out_shape=jax.ShapeDtypeStruct((P, N_EMBD), jnp.float32),
    )(bmapA, bmapB, xs, ws1.reshape(NB, 1, BLK), ws2.reshape(NB, 1, BLK),
      W1, W1, b1.reshape(NUM_EXPERTS, 1, D_FF), b1.reshape(NUM_EXPERTS, 1, D_FF),
      W2, W2, b2.reshape(NUM_EXPERTS, 1, N_EMBD),
      b2.reshape(NUM_EXPERTS, 1, N_EMBD))

    out = _run_combine(y, pos, T)
    return out.reshape(x.shape), loss[0, 0]


# speedup vs baseline: 1.1810x; 1.1516x over previous
import functools
import math

import jax
import jax.numpy as jnp
from jax import lax
from jax.experimental import pallas as pl
from jax.experimental.pallas import tpu as pltpu
from jax.experimental.pallas import tpu_sc as plsc

N_EMBD = 512
NUM_EXPERTS = 4
D_FF = 4 * N_EMBD
_GELU_C = math.sqrt(2.0 / math.pi)

BT = 1024
BLK = 512
NCOMBO = 6
_COMBO_A = (0, 0, 0, 1, 1, 2)
_COMBO_B = (1, 2, 3, 2, 3, 3)

_NC, _NS, _L = 2, 16, 16
_NW = _NC * _NS
_CH = 64


def _gelu(h):
    return 0.5 * h * (1.0 + jnp.tanh(_GELU_C * (h + 0.044715 * h ** 3)))


def _gate_body(x_ref, Wg_ref, c_ref, r_ref, wa_ref, wb_ref, tot_ref,
               loss_ref, gsum_ref, ut_ref):
    i = pl.program_id(0)
    nb = pl.num_programs(0)
    xb = x_ref[...]
    gT = lax.dot_general(Wg_ref[...], xb,
                         (((0,), (1,)), ((), ())))

    @pl.when(i == 0)
    def _():
        gsum_ref[0, 0] = 0.0
        tot_ref[...] = jnp.zeros_like(tot_ref)
        row_i = lax.broadcasted_iota(jnp.int32, (BT, BT), 0)
        col_i = lax.broadcasted_iota(jnp.int32, (BT, BT), 1)
        ut_ref[...] = (row_i < col_i).astype(jnp.float32)

    gsum_ref[0, 0] += jnp.sum(gT)

    iotaS = lax.broadcasted_iota(jnp.int32, gT.shape, 0)
    m1 = jnp.max(gT, axis=0, keepdims=True)
    e1 = jnp.min(jnp.where(gT == m1, iotaS, NUM_EXPERTS), axis=0,
                 keepdims=True)
    gm = jnp.where(iotaS == e1, -jnp.inf, gT)
    m2 = jnp.max(gm, axis=0, keepdims=True)
    e2 = jnp.min(jnp.where(gm == m2, iotaS, NUM_EXPERTS), axis=0,
                 keepdims=True)
    t = jnp.exp(m2 - m1)
    w1 = 1.0 / (1.0 + t)
    w2 = 1.0 - w1

    a = jnp.minimum(e1, e2)
    b = jnp.maximum(e1, e2)
    wa = jnp.where(e1 < e2, w1, w2)
    wb = jnp.where(e1 < e2, w2, w1)
    c = a * 3 - (a * (a - 1)) // 2 + (b - a - 1)

    crow = lax.broadcasted_iota(jnp.int32, (NCOMBO, BT), 0)
    onehotT = (c == crow).astype(jnp.float32)
    prevT = jnp.dot(onehotT, ut_ref[...])

    totT = tot_ref[...]
    r = jnp.sum((prevT + totT) * onehotT, axis=0, keepdims=True)
    tot_ref[...] = totT + jnp.sum(onehotT, axis=1, keepdims=True)

    c_ref[...] = c[None]
    r_ref[...] = r.astype(jnp.int32)[None]
    wa_ref[...] = wa[None]
    wb_ref[...] = wb[None]

    @pl.when(i == nb - 1)
    def _():
        s = gsum_ref[0, 0] / jnp.float32(nb * BT * NUM_EXPERTS)
        loss_ref[...] = jnp.broadcast_to(s * jnp.log(s + 0.1), (1, 1))


def _sc_mesh():
    return plsc.VectorSubcoreMesh(core_axis_name="c", subcore_axis_name="s",
                                  num_cores=_NC, num_subcores=_NS)


def _run_dispatch(flat, c, r, wa, wb, offp8, T, P):
    per = T // _NW
    nch = per // _CH

    @functools.partial(
        pl.kernel, mesh=_sc_mesh(),
        out_type=[
            jax.ShapeDtypeStruct((P, N_EMBD), jnp.float32),
            jax.ShapeDtypeStruct((P,), jnp.float32),
            jax.ShapeDtypeStruct((P,), jnp.float32),
            jax.ShapeDtypeStruct((_NW, nch, _CH), jnp.int32),
        ],
        scratch_types=[
            pltpu.VMEM((per,), jnp.int32),
            pltpu.VMEM((per,), jnp.int32),
            pltpu.VMEM((per,), jnp.float32),
            pltpu.VMEM((per,), jnp.float32),
            pltpu.VMEM((nch, _CH), jnp.int32),
            pltpu.VMEM((_CH, N_EMBD), jnp.float32),
            pltpu.VMEM((_CH, N_EMBD), jnp.float32),
            pltpu.VMEM((16,), jnp.int32),
            pltpu.SemaphoreType.DMA,
            pltpu.SemaphoreType.DMA,
            pltpu.SemaphoreType.DMA,
            pltpu.SemaphoreType.DMA,
        ],
    )
    def _dispatch(x_hbm, c_hbm, r_hbm, wa_hbm, wb_hbm, offp_hbm,
                  xs_hbm, ws1_hbm, ws2_hbm, pos_hbm,
                  c_v, r_v, wa_v, wb_v, p_v,
                  rows_v0, rows_v1, offp_v, semi0, semi1, semo0, semo1):
        wid = lax.axis_index("s") * _NC + lax.axis_index("c")
        base = wid * per
        pltpu.sync_copy(offp_hbm, offp_v)
        pltpu.sync_copy(c_hbm.at[pl.ds(base, per)], c_v)
        pltpu.sync_copy(r_hbm.at[pl.ds(base, per)], r_v)
        pltpu.sync_copy(wa_hbm.at[pl.ds(base, per)], wa_v)
        pltpu.sync_copy(wb_hbm.at[pl.ds(base, per)], wb_v)
        ov = offp_v[...]
        o0 = ov[0]
        o1 = ov[1]
        o2 = ov[2]
        o3 = ov[3]
        o4 = ov[4]
        o5 = ov[5]
        for jc in range(nch):
            for j in range(_CH // _L):
                sl = pl.ds(jc * _CH + j * _L, _L)
                cc = c_v[sl]
                off = jnp.where(
                    cc < 3,
                    jnp.where(cc == 0, o0, jnp.where(cc == 1, o1, o2)),
                    jnp.where(cc == 3, o3, jnp.where(cc == 4, o4, o5)))
                p_v[jc, pl.ds(j * _L, _L)] = off + r_v[sl]
        pltpu.sync_copy(p_v, pos_hbm.at[wid])

        rows_b = (rows_v0, rows_v1)
        semi_b = (semi0, semi1)
        semo_b = (semo0, semo1)
        pend_in = [None, None]
        pend_out = [None, None]

        def start_in(jc, bsel):
            o = base + jc * _CH
            pend_in[bsel] = pltpu.async_copy(
                x_hbm.at[pl.ds(o, _CH)], rows_b[bsel], semi_b[bsel])

        start_in(0, 0)
        for jc in range(nch):
            bsel = jc % 2
            if jc + 1 < nch:
                if pend_out[1 - bsel] is not None:
                    for cp in pend_out[1 - bsel]:
                        cp.wait()
                    pend_out[1 - bsel] = None
                start_in(jc + 1, 1 - bsel)
            o = base + jc * _CH
            pend_in[bsel].wait()
            pend_in[bsel] = None
            pend_out[bsel] = [
                pltpu.async_copy(rows_b[bsel], xs_hbm.at[p_v.at[jc]],
                                 semo_b[bsel]),
                pltpu.async_copy(wa_v.at[pl.ds(jc * _CH, _CH)],
                                 ws1_hbm.at[p_v.at[jc]], semo_b[bsel]),
                pltpu.async_copy(wb_v.at[pl.ds(jc * _CH, _CH)],
                                 ws2_hbm.at[p_v.at[jc]], semo_b[bsel]),
            ]
        for bsel in (0, 1):
            if pend_out[bsel] is not None:
                for cp in pend_out[bsel]:
                    cp.wait()

    return _dispatch(flat, c, r, wa, wb, offp8)


def _run_combine(y, pos, T):
    per = T // _NW
    nch = per // _CH

    @functools.partial(
        pl.kernel, mesh=_sc_mesh(),
        out_type=jax.ShapeDtypeStruct((T, N_EMBD), jnp.float32),
        scratch_types=[
            pltpu.VMEM((nch, _CH), jnp.int32),
            pltpu.VMEM((_CH, N_EMBD), jnp.float32),
            pltpu.VMEM((_CH, N_EMBD), jnp.float32),
            pltpu.SemaphoreType.DMA,
            pltpu.SemaphoreType.DMA,
            pltpu.SemaphoreType.DMA,
            pltpu.SemaphoreType.DMA,
        ],
    )
    def _combine(ys_hbm, pos_hbm, out_hbm, p_v, rows_v0, rows_v1,
                 semg0, semg1, semo0, semo1):
        wid = lax.axis_index("s") * _NC + lax.axis_index("c")
        base = wid * per
        pltpu.sync_copy(pos_hbm.at[wid], p_v)
        rows_b = (rows_v0, rows_v1)
        semg_b = (semg0, semg1)
        semo_b = (semo0, semo1)
        pend_g = [None, None]
        pend_o = [None, None]

        def start_gather(jc, bsel):
            pend_g[bsel] = pltpu.async_copy(ys_hbm.at[p_v.at[jc]],
                                            rows_b[bsel], semg_b[bsel])

        start_gather(0, 0)
        for jc in range(nch):
            bsel = jc % 2
            if jc + 1 < nch:
                if pend_o[1 - bsel] is not None:
                    pend_o[1 - bsel].wait()
                    pend_o[1 - bsel] = None
                start_gather(jc + 1, 1 - bsel)
            o = base + jc * _CH
            pend_g[bsel].wait()
            pend_g[bsel] = None
            pend_o[bsel] = pltpu.async_copy(
                rows_b[bsel], out_hbm.at[pl.ds(o, _CH)], semo_b[bsel])
        for bsel in (0, 1):
            if pend_o[bsel] is not None:
                pend_o[bsel].wait()

    return _combine(y, pos)


def _ffn_body(bmapA, bmapB, xs_ref, ws1_ref, ws2_ref, W1a_ref, W1b_ref,
              b1a_ref, b1b_ref, W2a_ref, W2b_ref, b2a_ref, b2b_ref, y_ref):
    del bmapA, bmapB
    xb = xs_ref[...]
    ha = _gelu(jnp.dot(xb, W1a_ref[0]) + b1a_ref[0, 0][None, :])
    ya = jnp.dot(ha, W2a_ref[0]) + b2a_ref[0, 0][None, :]
    hb = _gelu(jnp.dot(xb, W1b_ref[0]) + b1b_ref[0, 0][None, :])
    yb = jnp.dot(hb, W2b_ref[0]) + b2b_ref[0, 0][None, :]
    y_ref[...] = ws1_ref[0, 0][:, None] * ya + ws2_ref[0, 0][:, None] * yb


def kernel(x, Wg, W1, b1, W2, b2):
    T = x.shape[0] * x.shape[1]
    flat = x.reshape(T, N_EMBD)
    nb = T // BT
    NB = T // BLK + NCOMBO
    P = NB * BLK

    c3, r3, wa3, wb3, tot, loss = pl.pallas_call(
        _gate_body,
        grid=(nb,),
        in_specs=[
            pl.BlockSpec((BT, N_EMBD), lambda i: (i, 0)),
            pl.BlockSpec((N_EMBD, NUM_EXPERTS), lambda i: (0, 0)),
        ],
        out_specs=[
            pl.BlockSpec((1, 1, BT), lambda i: (i, 0, 0)),
            pl.BlockSpec((1, 1, BT), lambda i: (i, 0, 0)),
            pl.BlockSpec((1, 1, BT), lambda i: (i, 0, 0)),
            pl.BlockSpec((1, 1, BT), lambda i: (i, 0, 0)),
            pl.BlockSpec((NCOMBO, 1), lambda i: (0, 0)),
            pl.BlockSpec((1, 1), lambda i: (0, 0)),
        ],
        out_shape=[
            jax.ShapeDtypeStruct((nb, 1, BT), jnp.int32),
            jax.ShapeDtypeStruct((nb, 1, BT), jnp.int32),
            jax.ShapeDtypeStruct((nb, 1, BT), jnp.float32),
            jax.ShapeDtypeStruct((nb, 1, BT), jnp.float32),
            jax.ShapeDtypeStruct((NCOMBO, 1), jnp.float32),
            jax.ShapeDtypeStruct((1, 1), jnp.float32),
        ],
        scratch_shapes=[pltpu.SMEM((1, 1), jnp.float32),
                        pltpu.VMEM((BT, BT), jnp.float32)],
    )(flat, Wg)

    c = c3.reshape(T)
    r = r3.reshape(T)
    wa = wa3.reshape(T)
    wb = wb3.reshape(T)

    n = tot.reshape(NCOMBO).astype(jnp.int32)
    nblk = (n + BLK - 1) // BLK
    cumblk = jnp.cumsum(nblk)
    offp = (jnp.concatenate([jnp.zeros((1,), jnp.int32), cumblk[:-1]])
            * BLK).astype(jnp.int32)
    offp8 = jnp.pad(offp, (0, 16 - NCOMBO))
    bidx = jnp.arange(NB, dtype=jnp.int32)
    cb = jnp.minimum(
        jnp.sum((bidx[:, None] >= cumblk[None, :]).astype(jnp.int32), axis=1),
        NCOMBO - 1)
    bmapA = jnp.asarray(_COMBO_A, jnp.int32)[cb]
    bmapB = jnp.asarray(_COMBO_B, jnp.int32)[cb]

    xs, ws1, ws2, pos = _run_dispatch(flat, c, r, wa, wb, offp8, T, P)
    y = pl.pallas_call(
        _ffn_body,
        grid_spec=pltpu.PrefetchScalarGridSpec(
            num_scalar_prefetch=2,
            grid=(NB,),
            in_specs=[
                pl.BlockSpec((BLK, N_EMBD), lambda i, A, B: (i, 0)),
                pl.BlockSpec((1, 1, BLK), lambda i, A, B: (i, 0, 0)),
                pl.BlockSpec((1, 1, BLK), lambda i, A, B: (i, 0, 0)),
                pl.BlockSpec((1, N_EMBD, D_FF), lambda i, A, B: (A[i], 0, 0)),
                pl.BlockSpec((1, N_EMBD, D_FF), lambda i, A, B: (B[i], 0, 0)),
                pl.BlockSpec((1, 1, D_FF), lambda i, A, B: (A[i], 0, 0)),
                pl.BlockSpec((1, 1, D_FF), lambda i, A, B: (B[i], 0, 0)),
                pl.BlockSpec((1, D_FF, N_EMBD), lambda i, A, B: (A[i], 0, 0)),
                pl.BlockSpec((1, D_FF, N_EMBD), lambda i, A, B: (B[i], 0, 0)),
                pl.BlockSpec((1, 1, N_EMBD), lambda i, A, B: (A[i], 0, 0)),
                pl.BlockSpec((1, 1, N_EMBD), lambda i, A, B: (B[i], 0, 0)),
            ],
            out_specs=pl.BlockSpec((BLK, N_EMBD), lambda i, A, B: (i, 0)),
        ),
        out_shape=jax.ShapeDtypeStruct((P, N_EMBD), jnp.float32),
    )(bmapA, bmapB, xs, ws1.reshape(NB, 1, BLK), ws2.reshape(NB, 1, BLK),
      W1, W1, b1.reshape(NUM_EXPERTS, 1, D_FF),
      b1.reshape(NUM_EXPERTS, 1, D_FF),
      W2, W2, b2.reshape(NUM_EXPERTS, 1, N_EMBD),
      b2.reshape(NUM_EXPERTS, 1, N_EMBD))

    out = _run_combine(y, pos, T)
    return out.reshape(x.shape), loss[0, 0]
